# Initial kernel scaffold; baseline (speedup 1.0000x reference)
import jax
import jax.numpy as jnp
from jax import lax
from jax.experimental import pallas as pl
from jax.experimental.pallas import tpu as pltpu
from jax.experimental.pallas import tpu_sc as plsc

NUM_SEGMENTS = 16384
TOTAL_ATOMS = 524288
TABLE_PAD = 128

NC = 2
NS = 16
NW = NC * NS
CHUNK = TOTAL_ATOMS // NW
STRIPE = NUM_SEGMENTS // NS


def _sc_partials(atomic_numbers, segment_ids, table_padded):
    mesh = plsc.VectorSubcoreMesh(core_axis_name="c", subcore_axis_name="s")

    def body(an_hbm, seg_hbm, tab_hbm, out_hbm,
             an_v, seg_v, e_v, tab_v, zero_v, acc_sh):
        cid = lax.axis_index("c")
        sid = lax.axis_index("s")
        wid = cid * NS + sid
        base = wid * CHUNK

        for i in range(STRIPE // 16):
            zero_v[pl.ds(16 * i, 16)] = jnp.zeros((16,), jnp.float32)
        pltpu.sync_copy(zero_v, acc_sh.at[pl.ds(sid * STRIPE, STRIPE)])

        pltpu.sync_copy(an_hbm.at[pl.ds(base, CHUNK)], an_v)
        pltpu.sync_copy(seg_hbm.at[pl.ds(base, CHUNK)], seg_v)
        pltpu.sync_copy(tab_hbm, tab_v)

        pltpu.sync_copy(tab_v.at[an_v], e_v)

        plsc.subcore_barrier()
        pltpu.sync_copy(e_v, acc_sh.at[seg_v], add=True)
        plsc.subcore_barrier()

        pltpu.sync_copy(acc_sh.at[pl.ds(sid * STRIPE, STRIPE)],
                        out_hbm.at[cid, pl.ds(sid * STRIPE, STRIPE)])

    run = pl.kernel(
        body,
        out_type=jax.ShapeDtypeStruct((NC, NUM_SEGMENTS), jnp.float32),
        mesh=mesh,
        scratch_types=[
            pltpu.VMEM((CHUNK,), jnp.int32),
            pltpu.VMEM((CHUNK,), jnp.int32),
            pltpu.VMEM((CHUNK,), jnp.float32),
            pltpu.VMEM((TABLE_PAD,), jnp.float32),
            pltpu.VMEM((STRIPE,), jnp.float32),
            pltpu.VMEM_SHARED((NUM_SEGMENTS,), jnp.float32),
        ],
    )
    return run(atomic_numbers, segment_ids, table_padded)


def _merge_body(p_ref, o_ref):
    o_ref[...] = p_ref[0] + p_ref[1]


def _tc_merge(partials):
    p3 = partials.reshape(NC, 128, NUM_SEGMENTS // 128)
    out = pl.pallas_call(
        _merge_body,
        out_shape=jax.ShapeDtypeStruct((128, NUM_SEGMENTS // 128), jnp.float32),
    )(p3)
    return out.reshape(NUM_SEGMENTS, 1)


def kernel(atomic_numbers, segment_ids, property_per_element_table):
    table_padded = jnp.zeros((TABLE_PAD,), jnp.float32).at[
        :property_per_element_table.shape[0]].set(property_per_element_table)
    partials = _sc_partials(atomic_numbers, segment_ids, table_padded)
    return _tc_merge(partials)


# speedup vs baseline: 153.1689x; 153.1689x over previous
import jax
import jax.numpy as jnp
from jax import lax
from jax.experimental import pallas as pl
from jax.experimental.pallas import tpu as pltpu
from jax.experimental.pallas import tpu_sc as plsc

NUM_SEGMENTS = 16384
TOTAL_ATOMS = 524288
TABLE_PAD = 128

NC = 2
NS = 16
NW = NC * NS
CHUNK = TOTAL_ATOMS // NW
STRIPE = NUM_SEGMENTS // NS


def _sc_partials(atomic_numbers, segment_ids, table_padded):
    mesh = plsc.VectorSubcoreMesh(core_axis_name="c", subcore_axis_name="s")

    def body(an_hbm, seg_hbm, tab_hbm, out_hbm,
             an_v, seg_v, e_v, zero_v, tab_sh, acc_sh):
        cid = lax.axis_index("c")
        sid = lax.axis_index("s")
        wid = cid * NS + sid
        base = wid * CHUNK

        for i in range(STRIPE // 16):
            zero_v[pl.ds(16 * i, 16)] = jnp.zeros((16,), jnp.float32)
        pltpu.sync_copy(zero_v, acc_sh.at[pl.ds(sid * STRIPE, STRIPE)])

        pltpu.sync_copy(an_hbm.at[pl.ds(base, CHUNK)], an_v)
        pltpu.sync_copy(seg_hbm.at[pl.ds(base, CHUNK)], seg_v)

        @pl.when(sid == 0)
        def _():
            pltpu.sync_copy(tab_hbm, tab_sh)

        plsc.subcore_barrier()

        pltpu.sync_copy(tab_sh.at[an_v], e_v)

        pltpu.sync_copy(e_v, acc_sh.at[seg_v], add=True)
        plsc.subcore_barrier()

        pltpu.sync_copy(acc_sh.at[pl.ds(sid * STRIPE, STRIPE)],
                        out_hbm.at[cid, pl.ds(sid * STRIPE, STRIPE)])

    run = pl.kernel(
        body,
        out_type=jax.ShapeDtypeStruct((NC, NUM_SEGMENTS), jnp.float32),
        mesh=mesh,
        scratch_types=[
            pltpu.VMEM((CHUNK,), jnp.int32),
            pltpu.VMEM((CHUNK,), jnp.int32),
            pltpu.VMEM((CHUNK,), jnp.float32),
            pltpu.VMEM((STRIPE,), jnp.float32),
            pltpu.VMEM_SHARED((TABLE_PAD,), jnp.float32),
            pltpu.VMEM_SHARED((NUM_SEGMENTS,), jnp.float32),
        ],
    )
    return run(atomic_numbers, segment_ids, table_padded)


def _merge_body(p_ref, o_ref):
    o_ref[...] = p_ref[0] + p_ref[1]


def _tc_merge(partials):
    p3 = partials.reshape(NC, 128, NUM_SEGMENTS // 128)
    out = pl.pallas_call(
        _merge_body,
        out_shape=jax.ShapeDtypeStruct((128, NUM_SEGMENTS // 128), jnp.float32),
    )(p3)
    return out.reshape(NUM_SEGMENTS, 1)


def kernel(atomic_numbers, segment_ids, property_per_element_table):
    table_padded = jnp.zeros((TABLE_PAD,), jnp.float32).at[
        :property_per_element_table.shape[0]].set(property_per_element_table)
    partials = _sc_partials(atomic_numbers, segment_ids, table_padded)
    return _tc_merge(partials)


# speedup vs baseline: 176.4630x; 1.1521x over previous
import jax
import jax.numpy as jnp
from jax import lax
from jax.experimental import pallas as pl
from jax.experimental.pallas import tpu as pltpu
from jax.experimental.pallas import tpu_sc as plsc

NUM_SEGMENTS = 16384
TOTAL_ATOMS = 524288
TABLE_N = 95

NC = 2
NS = 16
NW = NC * NS
CHUNK = TOTAL_ATOMS // NW
NSUB = 8
SUB = CHUNK // NSUB
STRIPE = NUM_SEGMENTS // NS


def _sc_partials(atomic_numbers, segment_ids, table):
    mesh = plsc.VectorSubcoreMesh(core_axis_name="c", subcore_axis_name="s")

    def body(an_hbm, seg_hbm, tab_hbm, out_hbm,
             an_v, seg_v, e_v, zero_v, tab_v, acc_sh, sem_in, sem_sc):
        cid = lax.axis_index("c")
        sid = lax.axis_index("s")
        wid = cid * NS + sid
        base = wid * CHUNK

        in_copies = [
            pltpu.async_copy(an_hbm.at[pl.ds(base, CHUNK)], an_v, sem_in),
            pltpu.async_copy(seg_hbm.at[pl.ds(base, CHUNK)], seg_v, sem_in),
            pltpu.async_copy(tab_hbm, tab_v, sem_in),
        ]

        @plsc.parallel_loop(0, STRIPE // 16, unroll=4)
        def _(i):
            zero_v[pl.ds(pl.multiple_of(i * 16, 16), 16)] = (
                jnp.zeros((16,), jnp.float32))

        pltpu.sync_copy(zero_v, acc_sh.at[pl.ds(sid * STRIPE, STRIPE)])
        for cp in in_copies:
            cp.wait()
        plsc.subcore_barrier()

        scatters = []
        for j in range(NSUB):
            @plsc.parallel_loop(0, SUB // 16, unroll=4)
            def _(k, j=j):
                off = pl.multiple_of(j * SUB + k * 16, 16)
                an16 = an_v[pl.ds(off, 16)]
                e_v[pl.ds(off, 16)] = plsc.load_gather(tab_v, [an16])

            scatters.append(pltpu.async_copy(
                e_v.at[pl.ds(j * SUB, SUB)],
                acc_sh.at[seg_v.at[pl.ds(j * SUB, SUB)]],
                sem_sc, add=True))
        for dsc in scatters:
            dsc.wait()

        plsc.subcore_barrier()
        pltpu.sync_copy(acc_sh.at[pl.ds(sid * STRIPE, STRIPE)],
                        out_hbm.at[cid, pl.ds(sid * STRIPE, STRIPE)])

    run = pl.kernel(
        body,
        out_type=jax.ShapeDtypeStruct((NC, NUM_SEGMENTS), jnp.float32),
        mesh=mesh,
        scratch_types=[
            pltpu.VMEM((CHUNK,), jnp.int32),
            pltpu.VMEM((CHUNK,), jnp.int32),
            pltpu.VMEM((CHUNK,), jnp.float32),
            pltpu.VMEM((STRIPE,), jnp.float32),
            pltpu.VMEM((TABLE_N,), jnp.float32),
            pltpu.VMEM_SHARED((NUM_SEGMENTS,), jnp.float32),
            pltpu.SemaphoreType.DMA,
            pltpu.SemaphoreType.DMA,
        ],
        compiler_params=pltpu.CompilerParams(needs_layout_passes=False),
    )
    return run(atomic_numbers, segment_ids, table)


def _merge_body(p_ref, o_ref):
    o_ref[...] = p_ref[0] + p_ref[1]


def _tc_merge(partials):
    p3 = partials.reshape(NC, 128, NUM_SEGMENTS // 128)
    out = pl.pallas_call(
        _merge_body,
        out_shape=jax.ShapeDtypeStruct((128, NUM_SEGMENTS // 128), jnp.float32),
    )(p3)
    return out.reshape(NUM_SEGMENTS, 1)


def kernel(atomic_numbers, segment_ids, property_per_element_table):
    partials = _sc_partials(atomic_numbers, segment_ids,
                            property_per_element_table)
    return _tc_merge(partials)


# speedup vs baseline: 188.1650x; 1.0663x over previous
import jax
import jax.numpy as jnp
from jax import lax
from jax.experimental import pallas as pl
from jax.experimental.pallas import tpu as pltpu
from jax.experimental.pallas import tpu_sc as plsc

NUM_SEGMENTS = 16384
TOTAL_ATOMS = 524288
TABLE_N = 95

NC = 2
NS = 16
NW = NC * NS
CHUNK = TOTAL_ATOMS // NW
HALF = NUM_SEGMENTS // NC
OSTRIPE = HALF // NS


def _sc_kernel(atomic_numbers, segment_ids, table):
    mesh = plsc.VectorSubcoreMesh(core_axis_name="c", subcore_axis_name="s")

    def body(an_hbm, seg_hbm, tab_hbm, out_hbm,
             an_v, seg_v, tab_v, acc_v, pfirst_v, plast_v,
             tmp_v, sum_v, stage_sh, sem_in):
        cid = lax.axis_index("c")
        sid = lax.axis_index("s")
        lo = cid * HALF
        my_base = (cid * NS + sid) * CHUNK
        other_base = ((1 - cid) * NS + sid) * CHUNK

        in_copies = [
            pltpu.async_copy(an_hbm.at[pl.ds(my_base, CHUNK)], an_v, sem_in),
            pltpu.async_copy(seg_hbm.at[pl.ds(my_base, CHUNK)],
                             seg_v.at[pl.ds(0, CHUNK)], sem_in),
            pltpu.async_copy(tab_hbm, tab_v, sem_in),
            pltpu.async_copy(seg_hbm.at[pl.ds(other_base, 16)], pfirst_v,
                             sem_in),
            pltpu.async_copy(seg_hbm.at[pl.ds(other_base + CHUNK - 16, 16)],
                             plast_v, sem_in),
        ]

        @plsc.parallel_loop(0, NUM_SEGMENTS // 16, unroll=8)
        def _(i):
            acc_v[pl.ds(pl.multiple_of(i * 16, 16), 16)] = (
                jnp.zeros((16,), jnp.float32))

        seg_v[pl.ds(CHUNK, 16)] = jnp.full((16,), NUM_SEGMENTS - 1, jnp.int32)
        for cp in in_copies:
            cp.wait()

        lane = lax.iota(jnp.int32, 16)
        is15 = lane == 15

        def process_chunk():
            @plsc.parallel_loop(0, CHUNK // 16, unroll=4)
            def _(k):
                off = pl.multiple_of(k * 16, 16)
                an16 = an_v[pl.ds(off, 16)]
                seg = seg_v[pl.ds(off, 16)]
                segn = seg_v[pl.ds(off + 1, 16)]
                e = plsc.load_gather(tab_v, [an16])
                c = plsc.cumsum(e)
                m_change = seg != segn
                plsc.addupdate_scatter(acc_v, [seg], c,
                                       mask=m_change | is15)
                plsc.addupdate_scatter(acc_v, [segn], -c,
                                       mask=m_change & jnp.logical_not(is15))

        process_chunk()

        seg_first = pfirst_v[...][0]
        seg_last = plast_v[...][15]
        overlap = jnp.logical_and(seg_last >= lo, seg_first < lo + HALF)

        @pl.when(overlap)
        def _():
            pltpu.sync_copy(an_hbm.at[pl.ds(other_base, CHUNK)], an_v)
            pltpu.sync_copy(seg_hbm.at[pl.ds(other_base, CHUNK)],
                            seg_v.at[pl.ds(0, CHUNK)])
            process_chunk()

        pltpu.sync_copy(acc_v.at[pl.ds(lo, HALF)], stage_sh.at[sid])
        plsc.subcore_barrier()

        col = sid * OSTRIPE
        pltpu.sync_copy(stage_sh.at[0, pl.ds(col, OSTRIPE)], sum_v)
        for w in range(1, NS):
            pltpu.sync_copy(stage_sh.at[w, pl.ds(col, OSTRIPE)], tmp_v)

            @plsc.parallel_loop(0, OSTRIPE // 16, unroll=8)
            def _(i):
                off = pl.ds(pl.multiple_of(i * 16, 16), 16)
                sum_v[off] = sum_v[off] + tmp_v[off]

        pltpu.sync_copy(sum_v, out_hbm.at[pl.ds(lo + col, OSTRIPE)])

    run = pl.kernel(
        body,
        out_type=jax.ShapeDtypeStruct((NUM_SEGMENTS,), jnp.float32),
        mesh=mesh,
        scratch_types=[
            pltpu.VMEM((CHUNK,), jnp.int32),
            pltpu.VMEM((CHUNK + 16,), jnp.int32),
            pltpu.VMEM((TABLE_N,), jnp.float32),
            pltpu.VMEM((NUM_SEGMENTS,), jnp.float32),
            pltpu.VMEM((16,), jnp.int32),
            pltpu.VMEM((16,), jnp.int32),
            pltpu.VMEM((OSTRIPE,), jnp.float32),
            pltpu.VMEM((OSTRIPE,), jnp.float32),
            pltpu.VMEM_SHARED((NS, HALF), jnp.float32),
            pltpu.SemaphoreType.DMA,
        ],
        compiler_params=pltpu.CompilerParams(needs_layout_passes=False),
    )
    return run(atomic_numbers, segment_ids, table)


def kernel(atomic_numbers, segment_ids, property_per_element_table):
    out = _sc_kernel(atomic_numbers, segment_ids, property_per_element_table)
    return out.reshape(NUM_SEGMENTS, 1)


# speedup vs baseline: 204.9041x; 1.0890x over previous
import jax
import jax.numpy as jnp
from jax import lax
from jax.experimental import pallas as pl
from jax.experimental.pallas import tpu as pltpu
from jax.experimental.pallas import tpu_sc as plsc

NUM_SEGMENTS = 16384
TOTAL_ATOMS = 524288
TABLE_N = 95

NC = 2
NS = 16
NW = NC * NS
CHUNK = TOTAL_ATOMS // NW
HALF = NUM_SEGMENTS // NC
OSTRIPE = HALF // NS


def _sc_kernel(atomic_numbers, segment_ids, table):
    mesh = plsc.VectorSubcoreMesh(core_axis_name="c", subcore_axis_name="s")

    def body(an_hbm, seg_hbm, tab_hbm, out_hbm,
             an_v, seg_v, tab_v, acc_v, pfirst_v, plast_v,
             tmp_v, sum_v, stage_sh, sem_in):
        cid = lax.axis_index("c")
        sid = lax.axis_index("s")
        lo = cid * HALF
        my_base = (cid * NS + sid) * CHUNK
        other_base = ((1 - cid) * NS + sid) * CHUNK

        in_copies = [
            pltpu.async_copy(an_hbm.at[pl.ds(my_base, CHUNK)], an_v, sem_in),
            pltpu.async_copy(seg_hbm.at[pl.ds(my_base, CHUNK)],
                             seg_v.at[pl.ds(0, CHUNK)], sem_in),
            pltpu.async_copy(tab_hbm, tab_v, sem_in),
            pltpu.async_copy(seg_hbm.at[pl.ds(other_base, 16)], pfirst_v,
                             sem_in),
            pltpu.async_copy(seg_hbm.at[pl.ds(other_base + CHUNK - 16, 16)],
                             plast_v, sem_in),
        ]

        @plsc.parallel_loop(0, NUM_SEGMENTS // 16, unroll=8)
        def _(i):
            acc_v[pl.ds(pl.multiple_of(i * 16, 16), 16)] = (
                jnp.zeros((16,), jnp.float32))

        seg_v[pl.ds(CHUNK, 16)] = jnp.full((16,), NUM_SEGMENTS - 1, jnp.int32)
        for cp in in_copies:
            cp.wait()

        lane = lax.iota(jnp.int32, 16)
        is15 = lane == 15

        def do_vreg(off):
            an16 = an_v[pl.ds(off, 16)]
            seg = seg_v[pl.ds(off, 16)]
            segn = seg_v[pl.ds(off + 1, 16)]
            e = plsc.load_gather(tab_v, [an16])
            c = plsc.cumsum(e)
            m_change = seg != segn
            plsc.addupdate_scatter(acc_v, [seg], c,
                                   mask=m_change | is15)
            plsc.addupdate_scatter(acc_v, [segn], -c,
                                   mask=m_change & jnp.logical_not(is15))

        @plsc.parallel_loop(0, CHUNK // 16, unroll=4)
        def _(k):
            do_vreg(pl.multiple_of(k * 16, 16))

        seg_first = pfirst_v[...][0]
        seg_last = plast_v[...][15]
        hi = lo + HALF
        overlap = jnp.logical_and(seg_last >= lo, seg_first < hi)

        @pl.when(overlap)
        def _():
            pltpu.sync_copy(an_hbm.at[pl.ds(other_base, CHUNK)], an_v)
            pltpu.sync_copy(seg_hbm.at[pl.ds(other_base, CHUNK)],
                            seg_v.at[pl.ds(0, CHUNK)])

            def first_lane(k):
                return seg_v[pl.ds(k * 16, 16)][0]

            def search(pred):
                def step(_, ab):
                    a, b = ab
                    mid = (a + b) // 2
                    p = pred(mid)
                    return (jnp.where(p, a, mid + 1), jnp.where(p, mid, b))
                a, b = lax.fori_loop(0, 11, step,
                                     (jnp.int32(0), jnp.int32(CHUNK // 16)))
                return a

            klo = jnp.where(cid == 0, 0, search(lambda k: first_lane(k) >= lo))
            khi = jnp.where(cid == 0,
                            search(lambda k: first_lane(k) >= hi),
                            CHUNK // 16)
            klo = jnp.maximum(klo - 1, 0)
            khi = jnp.minimum(khi + 1, CHUNK // 16)

            def mbody(k, carry):
                do_vreg(k * 16)
                return carry

            lax.fori_loop(klo, khi, mbody, jnp.int32(0))

        pltpu.sync_copy(acc_v.at[pl.ds(lo, HALF)], stage_sh.at[sid])
        plsc.subcore_barrier()

        col = sid * OSTRIPE
        pltpu.sync_copy(stage_sh.at[0, pl.ds(col, OSTRIPE)], sum_v)
        for w in range(1, NS):
            pltpu.sync_copy(stage_sh.at[w, pl.ds(col, OSTRIPE)], tmp_v)

            @plsc.parallel_loop(0, OSTRIPE // 16, unroll=8)
            def _(i):
                off = pl.ds(pl.multiple_of(i * 16, 16), 16)
                sum_v[off] = sum_v[off] + tmp_v[off]

        pltpu.sync_copy(sum_v, out_hbm.at[pl.ds(lo + col, OSTRIPE)])

    run = pl.kernel(
        body,
        out_type=jax.ShapeDtypeStruct((NUM_SEGMENTS,), jnp.float32),
        mesh=mesh,
        scratch_types=[
            pltpu.VMEM((CHUNK,), jnp.int32),
            pltpu.VMEM((CHUNK + 16,), jnp.int32),
            pltpu.VMEM((TABLE_N,), jnp.float32),
            pltpu.VMEM((NUM_SEGMENTS,), jnp.float32),
            pltpu.VMEM((16,), jnp.int32),
            pltpu.VMEM((16,), jnp.int32),
            pltpu.VMEM((OSTRIPE,), jnp.float32),
            pltpu.VMEM((OSTRIPE,), jnp.float32),
            pltpu.VMEM_SHARED((NS, HALF), jnp.float32),
            pltpu.SemaphoreType.DMA,
        ],
        compiler_params=pltpu.CompilerParams(needs_layout_passes=False),
    )
    return run(atomic_numbers, segment_ids, table)


def kernel(atomic_numbers, segment_ids, property_per_element_table):
    out = _sc_kernel(atomic_numbers, segment_ids, property_per_element_table)
    return out.reshape(NUM_SEGMENTS, 1)


# speedup vs baseline: 226.7564x; 1.1066x over previous
import jax
import jax.numpy as jnp
from jax import lax
from jax.experimental import pallas as pl
from jax.experimental.pallas import tpu as pltpu
from jax.experimental.pallas import tpu_sc as plsc

NUM_SEGMENTS = 16384
TOTAL_ATOMS = 524288
TABLE_N = 95

NC = 2
NS = 16
NW = NC * NS
CHUNK = TOTAL_ATOMS // NW
KV = CHUNK // 16
HALF = NUM_SEGMENTS // NC
OSTRIPE = HALF // NS


def _sc_kernel(atomic_numbers, segment_ids, table):
    mesh = plsc.VectorSubcoreMesh(core_axis_name="c", subcore_axis_name="s")

    def body(an_hbm, seg_hbm, tab_hbm, out_hbm,
             an_v, seg_v, an2_v, seg2_v, tab_v, acc_v,
             tmp16_v, sum_v, stage_sh, sem_in, sem_m):
        cid = lax.axis_index("c")
        sid = lax.axis_index("s")
        lo = cid * HALF
        hi = lo + HALF
        my_base = (cid * NS + sid) * CHUNK
        other_base = ((1 - cid) * NS + sid) * CHUNK

        in_copies = [
            pltpu.async_copy(an_hbm.at[pl.ds(my_base, CHUNK)], an_v, sem_in),
            pltpu.async_copy(seg_hbm.at[pl.ds(my_base, CHUNK)],
                             seg_v.at[pl.ds(0, CHUNK)], sem_in),
            pltpu.async_copy(tab_hbm, tab_v, sem_in),
        ]
        m_copies = [
            pltpu.async_copy(an_hbm.at[pl.ds(other_base, CHUNK)], an2_v,
                             sem_m),
            pltpu.async_copy(seg_hbm.at[pl.ds(other_base, CHUNK)],
                             seg2_v.at[pl.ds(0, CHUNK)], sem_m),
        ]

        @plsc.parallel_loop(0, NUM_SEGMENTS // 16, unroll=8)
        def _(i):
            acc_v[pl.ds(pl.multiple_of(i * 16, 16), 16)] = (
                jnp.zeros((16,), jnp.float32))

        seg_v[pl.ds(CHUNK, 16)] = jnp.full((16,), NUM_SEGMENTS - 1, jnp.int32)
        seg2_v[pl.ds(CHUNK, 16)] = jnp.full((16,), NUM_SEGMENTS - 1, jnp.int32)
        for cp in in_copies:
            cp.wait()

        lane = lax.iota(jnp.int32, 16)
        is15 = lane == 15
        not15 = jnp.logical_not(is15)

        def do_vreg(an_ref, seg_ref, off):
            an16 = an_ref[pl.ds(off, 16)]
            seg = seg_ref[pl.ds(off, 16)]
            segn = seg_ref[pl.ds(off + 1, 16)]
            e = plsc.load_gather(tab_v, [an16])
            c = plsc.cumsum(e)
            m_change = seg != segn
            plsc.addupdate_scatter(acc_v, [seg], c, mask=m_change | is15)
            plsc.addupdate_scatter(acc_v, [segn], -c, mask=m_change & not15)

        @plsc.parallel_loop(0, KV, unroll=8)
        def _(k):
            do_vreg(an_v, seg_v, pl.multiple_of(k * 16, 16))

        for cp in m_copies:
            cp.wait()
        seg_first = seg2_v[pl.ds(0, 16)][0]
        seg_last = seg2_v[pl.ds(CHUNK - 16, 16)][15]
        overlap = jnp.logical_and(seg_last >= lo, seg_first < hi)

        @pl.when(overlap)
        def _():
            def first_lane(k):
                return seg2_v[pl.ds(k * 16, 16)][0]

            def search(bound):
                def step(_, ab):
                    a, b = ab
                    mid = (a + b) // 2
                    p = first_lane(mid) >= bound
                    return (jnp.where(p, a, mid + 1), jnp.where(p, mid, b))
                return lax.fori_loop(0, 11, step,
                                     (jnp.int32(0), jnp.int32(KV)))[0]

            klo = jnp.where(cid == 0, 0, jnp.maximum(search(lo) - 1, 0))
            khi = jnp.where(cid == 0, jnp.minimum(search(hi) + 1, KV), KV)

            def mbody(k, carry):
                do_vreg(an2_v, seg2_v, k * 16)
                return carry

            lax.fori_loop(klo, khi, mbody, jnp.int32(0))

        pltpu.sync_copy(acc_v.at[pl.ds(lo, HALF)], stage_sh.at[sid])
        plsc.subcore_barrier()

        col = sid * OSTRIPE
        r_copies = [
            pltpu.async_copy(stage_sh.at[w, pl.ds(col, OSTRIPE)],
                             tmp16_v.at[w], sem_in)
            for w in range(NS)
        ]
        for cp in r_copies:
            cp.wait()

        @plsc.parallel_loop(0, OSTRIPE // 16, unroll=8)
        def _(i):
            off = pl.ds(pl.multiple_of(i * 16, 16), 16)
            s = tmp16_v[0, off]
            for w in range(1, NS):
                s = s + tmp16_v[w, off]
            sum_v[off] = s

        pltpu.sync_copy(sum_v, out_hbm.at[pl.ds(lo + col, OSTRIPE)])

    run = pl.kernel(
        body,
        out_type=jax.ShapeDtypeStruct((NUM_SEGMENTS,), jnp.float32),
        mesh=mesh,
        scratch_types=[
            pltpu.VMEM((CHUNK,), jnp.int32),
            pltpu.VMEM((CHUNK + 16,), jnp.int32),
            pltpu.VMEM((CHUNK,), jnp.int32),
            pltpu.VMEM((CHUNK + 16,), jnp.int32),
            pltpu.VMEM((TABLE_N,), jnp.float32),
            pltpu.VMEM((NUM_SEGMENTS,), jnp.float32),
            pltpu.VMEM((NS, OSTRIPE), jnp.float32),
            pltpu.VMEM((OSTRIPE,), jnp.float32),
            pltpu.VMEM_SHARED((NS, HALF), jnp.float32),
            pltpu.SemaphoreType.DMA,
            pltpu.SemaphoreType.DMA,
        ],
        compiler_params=pltpu.CompilerParams(needs_layout_passes=False),
    )
    return run(atomic_numbers, segment_ids, table)


def kernel(atomic_numbers, segment_ids, property_per_element_table):
    out = _sc_kernel(atomic_numbers, segment_ids, property_per_element_table)
    return out.reshape(NUM_SEGMENTS, 1)


# speedup vs baseline: 232.6556x; 1.0260x over previous
import jax
import jax.numpy as jnp
from jax import lax
from jax.experimental import pallas as pl
from jax.experimental.pallas import tpu as pltpu
from jax.experimental.pallas import tpu_sc as plsc

NUM_SEGMENTS = 16384
TOTAL_ATOMS = 524288
TABLE_N = 95

NC = 2
NS = 16
NW = NC * NS
CHUNK = TOTAL_ATOMS // NW
KV = CHUNK // 16
HALF = NUM_SEGMENTS // NC
OSTRIPE = HALF // NS
SUBC = 4
SCH = CHUNK // SUBC


def _sc_kernel(atomic_numbers, segment_ids, table):
    mesh = plsc.VectorSubcoreMesh(core_axis_name="c", subcore_axis_name="s")

    def body(an_hbm, seg_hbm, tab_hbm, out_hbm,
             an_v, seg_v, an2_v, seg2_v, tab_v, acc_v,
             tmp16_v, sum_v, stage_sh, sem_in, sem_m, *sems):
        cid = lax.axis_index("c")
        sid = lax.axis_index("s")
        lo = cid * HALF
        hi = lo + HALF
        my_base = (cid * NS + sid) * CHUNK
        other_base = ((1 - cid) * NS + sid) * CHUNK

        tab_copy = pltpu.async_copy(tab_hbm, tab_v, sem_in)
        sub_copies = []
        for j in range(SUBC):
            o = j * SCH
            sub_copies.append((
                pltpu.async_copy(an_hbm.at[pl.ds(my_base + o, SCH)],
                                 an_v.at[pl.ds(o, SCH)], sems[j]),
                pltpu.async_copy(seg_hbm.at[pl.ds(my_base + o, SCH)],
                                 seg_v.at[pl.ds(o, SCH)], sems[j]),
            ))
        m_copies = [
            pltpu.async_copy(an_hbm.at[pl.ds(other_base, CHUNK)], an2_v,
                             sem_m),
            pltpu.async_copy(seg_hbm.at[pl.ds(other_base, CHUNK)],
                             seg2_v.at[pl.ds(0, CHUNK)], sem_m),
        ]

        @plsc.parallel_loop(0, NUM_SEGMENTS // 16, unroll=8)
        def _(i):
            acc_v[pl.ds(pl.multiple_of(i * 16, 16), 16)] = (
                jnp.zeros((16,), jnp.float32))

        seg_v[pl.ds(CHUNK, 16)] = jnp.full((16,), NUM_SEGMENTS - 1, jnp.int32)
        seg2_v[pl.ds(CHUNK, 16)] = jnp.full((16,), NUM_SEGMENTS - 1, jnp.int32)
        tab_copy.wait()

        lane = lax.iota(jnp.int32, 16)
        is15 = lane == 15
        not15 = jnp.logical_not(is15)

        def do_vreg(an_ref, seg_ref, off):
            an16 = an_ref[pl.ds(off, 16)]
            seg = seg_ref[pl.ds(off, 16)]
            segn = seg_ref[pl.ds(off + 1, 16)]
            e = plsc.load_gather(tab_v, [an16])
            c = plsc.cumsum(e)
            m_change = seg != segn
            plsc.addupdate_scatter(acc_v, [seg], c, mask=m_change | is15)
            plsc.addupdate_scatter(acc_v, [segn], -c, mask=m_change & not15)

        for j in range(SUBC):
            for cp in sub_copies[j]:
                cp.wait()

            @plsc.parallel_loop(0, SCH // 16, unroll=8)
            def _(k, j=j):
                do_vreg(an_v, seg_v,
                        pl.multiple_of(j * SCH + k * 16, 16))

        for cp in m_copies:
            cp.wait()
        seg_first = seg2_v[pl.ds(0, 16)][0]
        seg_last = seg2_v[pl.ds(CHUNK - 16, 16)][15]
        overlap = jnp.logical_and(seg_last >= lo, seg_first < hi)

        @pl.when(overlap)
        def _():
            def first_lane(k):
                return seg2_v[pl.ds(k * 16, 16)][0]

            def search(bound):
                def step(_, ab):
                    a, b = ab
                    mid = (a + b) // 2
                    p = first_lane(mid) >= bound
                    return (jnp.where(p, a, mid + 1), jnp.where(p, mid, b))
                return lax.fori_loop(0, 11, step,
                                     (jnp.int32(0), jnp.int32(KV)))[0]

            klo = jnp.where(cid == 0, 0, jnp.maximum(search(lo) - 1, 0))
            khi = jnp.where(cid == 0, jnp.minimum(search(hi) + 1, KV), KV)

            def mbody(k, carry):
                do_vreg(an2_v, seg2_v, k * 16)
                return carry

            lax.fori_loop(klo, khi, mbody, jnp.int32(0))

        pltpu.sync_copy(acc_v.at[pl.ds(lo, HALF)], stage_sh.at[sid])
        plsc.subcore_barrier()

        col = sid * OSTRIPE
        r_copies = [
            pltpu.async_copy(stage_sh.at[w, pl.ds(col, OSTRIPE)],
                             tmp16_v.at[w], sem_in)
            for w in range(NS)
        ]
        for cp in r_copies:
            cp.wait()

        @plsc.parallel_loop(0, OSTRIPE // 16, unroll=8)
        def _(i):
            off = pl.ds(pl.multiple_of(i * 16, 16), 16)
            s = tmp16_v[0, off]
            for w in range(1, NS):
                s = s + tmp16_v[w, off]
            sum_v[off] = s

        pltpu.sync_copy(sum_v, out_hbm.at[pl.ds(lo + col, OSTRIPE)])

    run = pl.kernel(
        body,
        out_type=jax.ShapeDtypeStruct((NUM_SEGMENTS,), jnp.float32),
        mesh=mesh,
        scratch_types=[
            pltpu.VMEM((CHUNK,), jnp.int32),
            pltpu.VMEM((CHUNK + 16,), jnp.int32),
            pltpu.VMEM((CHUNK,), jnp.int32),
            pltpu.VMEM((CHUNK + 16,), jnp.int32),
            pltpu.VMEM((TABLE_N,), jnp.float32),
            pltpu.VMEM((NUM_SEGMENTS,), jnp.float32),
            pltpu.VMEM((NS, OSTRIPE), jnp.float32),
            pltpu.VMEM((OSTRIPE,), jnp.float32),
            pltpu.VMEM_SHARED((NS, HALF), jnp.float32),
            pltpu.SemaphoreType.DMA,
            pltpu.SemaphoreType.DMA,
        ] + [pltpu.SemaphoreType.DMA] * SUBC,
        compiler_params=pltpu.CompilerParams(needs_layout_passes=False),
    )
    return run(atomic_numbers, segment_ids, table)


def kernel(atomic_numbers, segment_ids, property_per_element_table):
    out = _sc_kernel(atomic_numbers, segment_ids, property_per_element_table)
    return out.reshape(NUM_SEGMENTS, 1)
